# baseline (speedup 1.0000x reference)
import jax
import jax.numpy as jnp
from jax.experimental import pallas as pl
from jax.experimental.pallas import tpu as pltpu


def _mm_body(a_ref, w_ref, out_ref):
    out_ref[...] = a_ref[0, :256, :64] + w_ref[:256, :]


def kernel(adj, weight):
    m, k = adj.shape
    k2, n = weight.shape
    assert k == k2
    bm = 256
    adj4 = adj.reshape(m // bm, bm * k // 128, 128)
    grid = (m // bm,)
    return pl.pallas_call(
        _mm_body,
        grid=grid,
        in_specs=[
            pl.BlockSpec((1, bm * k // 128, 128), lambda i: (i, 0, 0)),
            pl.BlockSpec((k2, n), lambda i: (0, 0)),
        ],
        out_specs=pl.BlockSpec((bm, n), lambda i: (i, 0)),
        out_shape=jax.ShapeDtypeStruct((m, n), jnp.float32),
        compiler_params=pltpu.CompilerParams(
            dimension_semantics=("arbitrary",),
        ),
    )(adj4, weight)


# speedup vs baseline: 4.3456x; 4.3456x over previous
import functools

import jax
import jax.numpy as jnp
from jax.experimental import pallas as pl
from jax.experimental.pallas import tpu as pltpu


def _mm_body(adj_hbm, w_ref, out_ref, bufa, bufb, sema, semb, *, bm, nblocks):
    def cp(i, buf, sem):
        return pltpu.make_async_copy(
            adj_hbm.at[pl.ds(i * bm, bm), :], buf, sem
        )

    cp(0, bufa, sema).start()
    cp(1, bufb, semb).start()

    def half_step(i, buf, sem):
        cp(i, buf, sem).wait()
        out_ref[pl.ds(i * bm, bm), :] = jnp.dot(
            buf[...], w_ref[...], preferred_element_type=jnp.float32
        )

        @pl.when(i + 2 < nblocks)
        def _():
            cp(i + 2, buf, sem).start()

    def step(j, carry):
        half_step(2 * j, bufa, sema)
        half_step(2 * j + 1, bufb, semb)
        return carry

    jax.lax.fori_loop(0, nblocks // 2, step, 0)


def kernel(adj, weight):
    m, k = adj.shape
    k2, n = weight.shape
    assert k == k2
    bm = 256
    nblocks = m // bm
    return pl.pallas_call(
        functools.partial(_mm_body, bm=bm, nblocks=nblocks),
        in_specs=[
            pl.BlockSpec(memory_space=pltpu.HBM),
            pl.BlockSpec((k2, n), lambda: (0, 0)),
        ],
        out_specs=pl.BlockSpec((m, n), lambda: (0, 0)),
        out_shape=jax.ShapeDtypeStruct((m, n), jnp.float32),
        scratch_shapes=[
            pltpu.VMEM((bm, k), jnp.float32),
            pltpu.VMEM((bm, k), jnp.float32),
            pltpu.SemaphoreType.DMA,
            pltpu.SemaphoreType.DMA,
        ],
    )(adj, weight)


# speedup vs baseline: 4.5271x; 1.0418x over previous
import functools

import jax
import jax.numpy as jnp
from jax.experimental import pallas as pl
from jax.experimental.pallas import tpu as pltpu


def _mm_body(adj_hbm, w_ref, out_ref, bufa, bufb, sema, semb, *, bm, nblocks):
    def cp(i, buf, sem, slot):
        return pltpu.make_async_copy(
            adj_hbm.at[pl.ds(i * bm, bm), :], buf.at[slot], sem.at[slot]
        )

    cp(0, bufa, sema, 0).start()
    cp(1, bufb, semb, 0).start()
    cp(2, bufa, sema, 1).start()
    cp(3, bufb, semb, 1).start()

    def half_step(i, j, buf, sem):
        slot = jax.lax.rem(j, 2)
        cp(i, buf, sem, slot).wait()
        out_ref[pl.ds(i * bm, bm), :] = jnp.dot(
            buf[slot], w_ref[...], preferred_element_type=jnp.float32
        )

        @pl.when(i + 4 < nblocks)
        def _():
            cp(i + 4, buf, sem, slot).start()

    def step(j, carry):
        half_step(2 * j, j, bufa, sema)
        half_step(2 * j + 1, j, bufb, semb)
        return carry

    jax.lax.fori_loop(0, nblocks // 2, step, 0)


def kernel(adj, weight):
    m, k = adj.shape
    k2, n = weight.shape
    assert k == k2
    bm = 128
    nblocks = m // bm
    return pl.pallas_call(
        functools.partial(_mm_body, bm=bm, nblocks=nblocks),
        in_specs=[
            pl.BlockSpec(memory_space=pltpu.HBM),
            pl.BlockSpec((k2, n), lambda: (0, 0)),
        ],
        out_specs=pl.BlockSpec((m, n), lambda: (0, 0)),
        out_shape=jax.ShapeDtypeStruct((m, n), jnp.float32),
        scratch_shapes=[
            pltpu.VMEM((2, bm, k), jnp.float32),
            pltpu.VMEM((2, bm, k), jnp.float32),
            pltpu.SemaphoreType.DMA((2,)),
            pltpu.SemaphoreType.DMA((2,)),
        ],
    )(adj, weight)
